# baseline (speedup 1.0000x reference)
import functools

import jax
import jax.numpy as jnp
from jax import lax
from jax.experimental import pallas as pl
from jax.experimental.pallas import tpu as pltpu
from jax.experimental.pallas import tpu_sc as plsc

N_EDGES = 320000
D = 128
NC, NS, L = 2, 16, 16
NW = NC * NS
EPW = N_EDGES // NW
E = 80
NCHUNK = EPW // E

_mesh = plsc.VectorSubcoreMesh(
    core_axis_name="c", subcore_axis_name="s", num_cores=NC, num_subcores=NS)


@functools.partial(
    pl.kernel,
    mesh=_mesh,
    compiler_params=pltpu.CompilerParams(use_tc_tiling_on_sc=False,
                                         needs_layout_passes=False),
    out_type=jax.ShapeDtypeStruct((N_EDGES,), jnp.float32),
    scratch_types=[
        pltpu.VMEM((2, 3, E), jnp.int32),
        pltpu.VMEM((2, E, 2 * D), jnp.bfloat16),
        pltpu.VMEM((2, E, 2 * D), jnp.bfloat16),
        pltpu.VMEM((2, E, 2 * D), jnp.bfloat16),
        pltpu.VMEM((2, E), jnp.float32),
        pltpu.SemaphoreType.DMA,
        pltpu.SemaphoreType.DMA,
        pltpu.SemaphoreType.DMA,
        pltpu.SemaphoreType.DMA,
        pltpu.SemaphoreType.DMA,
        pltpu.SemaphoreType.DMA,
    ],
)
def _sc_score(xf, rf, idx_all, out,
              idx_v, s_v, o_v, r_v, out_v,
              sr0, sr1, si0, si1, so0, so1):
    wid = lax.axis_index("s") * NC + lax.axis_index("c")
    base0 = wid * EPW
    lanes = lax.iota(jnp.int32, L)
    mask0 = lanes < 1
    sem_rows = (sr0, sr1)
    sem_idx = (si0, si1)
    sem_out = (so0, so1)

    def stage_idx(b, ci, sync=False):
        @pl.when(ci < NCHUNK)
        def _():
            base = base0 + ci * E
            if sync:
                pltpu.sync_copy(idx_all.at[:, pl.ds(base, E)], idx_v.at[b])
            else:
                pltpu.async_copy(idx_all.at[:, pl.ds(base, E)], idx_v.at[b],
                                 sem_idx[b])

    def wait_idx(b, ci):
        @pl.when(ci < NCHUNK)
        def _():
            pltpu.make_async_copy(idx_all.at[:, pl.ds(base0, E)], idx_v.at[b],
                                  sem_idx[b]).wait()

    def stage_rows(b, ci):
        @pl.when(ci < NCHUNK)
        def _():
            pltpu.async_copy(xf.at[idx_v.at[b, 0]], s_v.at[b], sem_rows[b])
            pltpu.async_copy(xf.at[idx_v.at[b, 1]], o_v.at[b], sem_rows[b])
            pltpu.async_copy(rf.at[idx_v.at[b, 2]], r_v.at[b], sem_rows[b])

    def wait_rows(b):
        pltpu.make_async_copy(xf.at[idx_v.at[b, 0]], s_v.at[b],
                              sem_rows[b]).wait()
        pltpu.make_async_copy(xf.at[idx_v.at[b, 1]], o_v.at[b],
                              sem_rows[b]).wait()
        pltpu.make_async_copy(rf.at[idx_v.at[b, 2]], r_v.at[b],
                              sem_rows[b]).wait()

    def wait_out(b, ci_prev):
        @pl.when(ci_prev >= 0)
        def _():
            pltpu.make_async_copy(out_v.at[b], out.at[pl.ds(base0, E)],
                                  sem_out[b]).wait()

    def compute(b, ci):
        sb, ob, rb, outb = s_v.at[b], o_v.at[b], r_v.at[b], out_v.at[b]
        wait_out(b, ci - 2)

        HIMASK = jnp.int32(-65536)

        def halves(ref, off, e):
            v = plsc.bitcast(ref[e, pl.ds(off, 2 * L)], jnp.int32)
            lo = plsc.bitcast(v << 16, jnp.float32)
            hi = plsc.bitcast(v & HIMASK, jnp.float32)
            return lo, hi

        @plsc.parallel_loop(0, E, step=1, unroll=4)
        def edge_body(e):
            accs = [jnp.zeros((L,), jnp.float32) for _ in range(4)]
            for j in range(D // (2 * L)):
                aa = halves(sb, j * 2 * L, e)
                bb = halves(sb, D + j * 2 * L, e)
                cc = halves(rb, j * 2 * L, e)
                dd = halves(rb, D + j * 2 * L, e)
                ee = halves(ob, j * 2 * L, e)
                ff = halves(ob, D + j * 2 * L, e)
                for k in range(2):
                    a, b_, c, d_ = aa[k], bb[k], cc[k], dd[k]
                    e_, f_ = ee[k], ff[k]
                    accs[(2 * j + k) % 4] = accs[(2 * j + k) % 4] + (
                        e_ * (a * c - b_ * d_) - f_ * (a * d_ + b_ * c))
            acc = (accs[0] + accs[1]) + (accs[2] + accs[3])
            tot = jnp.full((L,), jnp.sum(acc), jnp.float32)
            plsc.store_scatter(outb, [jnp.full((L,), e, jnp.int32)], tot,
                               mask=mask0)
        pltpu.async_copy(outb, out.at[pl.ds(base0 + ci * E, E)], sem_out[b])

    stage_idx(0, 0, sync=True)
    stage_rows(0, 0)
    stage_idx(1, 1)

    def pair_body(p, carry):
        c0 = p * 2
        wait_rows(0)
        stage_idx(0, c0 + 2)
        wait_idx(1, c0 + 1)
        stage_rows(1, c0 + 1)
        compute(0, c0)
        wait_rows(1)
        stage_idx(1, c0 + 3)
        wait_idx(0, c0 + 2)
        stage_rows(0, c0 + 2)
        compute(1, c0 + 1)
        return carry

    lax.fori_loop(0, NCHUNK // 2, pair_body, 0)
    wait_rows(0)
    compute(0, NCHUNK - 1)
    wait_out(0, NCHUNK - 1)
    wait_out(1, NCHUNK - 2)


def kernel(x, edge_index, edge_type, R_diagonal):
    xf = jnp.concatenate([jnp.real(x), jnp.imag(x)], axis=1).astype(jnp.bfloat16)
    rf = jnp.concatenate(
        [jnp.real(R_diagonal), jnp.imag(R_diagonal)], axis=1).astype(jnp.bfloat16)
    idx_all = jnp.concatenate([edge_index, edge_type[None, :]], axis=0)
    return _sc_score(xf, rf, idx_all)


# speedup vs baseline: 1.7310x; 1.7310x over previous
import functools

import jax
import jax.numpy as jnp
from jax import lax
from jax.experimental import pallas as pl
from jax.experimental.pallas import tpu as pltpu
from jax.experimental.pallas import tpu_sc as plsc

N_EDGES = 320000
D = 128
NC, NS, L = 2, 16, 16
NW = NC * NS
EPW = N_EDGES // NW
E = 80
NCHUNK = EPW // E

_mesh = plsc.VectorSubcoreMesh(
    core_axis_name="c", subcore_axis_name="s", num_cores=NC, num_subcores=NS)


@functools.partial(
    pl.kernel,
    mesh=_mesh,
    compiler_params=pltpu.CompilerParams(use_tc_tiling_on_sc=False,
                                         needs_layout_passes=False),
    out_type=jax.ShapeDtypeStruct((N_EDGES,), jnp.float32),
    scratch_types=[
        pltpu.VMEM((2, 3, E), jnp.int32),
        pltpu.VMEM((2, E, 2 * D), jnp.float32),
        pltpu.VMEM((2, E, 2 * D), jnp.float32),
        pltpu.VMEM((2, E, 2 * D), jnp.float32),
        pltpu.VMEM((2, E), jnp.float32),
        pltpu.SemaphoreType.DMA,
        pltpu.SemaphoreType.DMA,
        pltpu.SemaphoreType.DMA,
        pltpu.SemaphoreType.DMA,
        pltpu.SemaphoreType.DMA,
        pltpu.SemaphoreType.DMA,
    ],
)
def _sc_score(xf, rf, idx_all, out,
              idx_v, s_v, o_v, r_v, out_v,
              sr0, sr1, si0, si1, so0, so1):
    wid = lax.axis_index("s") * NC + lax.axis_index("c")
    base0 = wid * EPW
    lanes = lax.iota(jnp.int32, L)
    mask0 = lanes < 1
    sem_rows = (sr0, sr1)
    sem_idx = (si0, si1)
    sem_out = (so0, so1)

    def stage_idx(b, ci, sync=False):
        @pl.when(ci < NCHUNK)
        def _():
            base = base0 + ci * E
            if sync:
                pltpu.sync_copy(idx_all.at[:, pl.ds(base, E)], idx_v.at[b])
            else:
                pltpu.async_copy(idx_all.at[:, pl.ds(base, E)], idx_v.at[b],
                                 sem_idx[b])

    def wait_idx(b, ci):
        @pl.when(ci < NCHUNK)
        def _():
            pltpu.make_async_copy(idx_all.at[:, pl.ds(base0, E)], idx_v.at[b],
                                  sem_idx[b]).wait()

    def stage_rows(b, ci):
        @pl.when(ci < NCHUNK)
        def _():
            pltpu.async_copy(xf.at[idx_v.at[b, 0]], s_v.at[b], sem_rows[b])
            pltpu.async_copy(xf.at[idx_v.at[b, 1]], o_v.at[b], sem_rows[b])
            pltpu.async_copy(rf.at[idx_v.at[b, 2]], r_v.at[b], sem_rows[b])

    def wait_rows(b):
        pltpu.make_async_copy(xf.at[idx_v.at[b, 0]], s_v.at[b],
                              sem_rows[b]).wait()
        pltpu.make_async_copy(xf.at[idx_v.at[b, 1]], o_v.at[b],
                              sem_rows[b]).wait()
        pltpu.make_async_copy(rf.at[idx_v.at[b, 2]], r_v.at[b],
                              sem_rows[b]).wait()

    def wait_out(b, ci_prev):
        @pl.when(ci_prev >= 0)
        def _():
            pltpu.make_async_copy(out_v.at[b], out.at[pl.ds(base0, E)],
                                  sem_out[b]).wait()

    def compute(b, ci):
        sb, ob, rb, outb = s_v.at[b], o_v.at[b], r_v.at[b], out_v.at[b]
        wait_out(b, ci - 2)

        @plsc.parallel_loop(0, E, step=1, unroll=4)
        def edge_body(e):
            accs = [jnp.zeros((L,), jnp.float32) for _ in range(4)]
            for j in range(D // L):
                a = sb[e, pl.ds(j * L, L)]
                b_ = sb[e, pl.ds(D + j * L, L)]
                c = rb[e, pl.ds(j * L, L)]
                d_ = rb[e, pl.ds(D + j * L, L)]
                e_ = ob[e, pl.ds(j * L, L)]
                f_ = ob[e, pl.ds(D + j * L, L)]
                accs[j % 4] = accs[j % 4] + (
                    e_ * (a * c - b_ * d_) - f_ * (a * d_ + b_ * c))
            acc = (accs[0] + accs[1]) + (accs[2] + accs[3])
            tot = jnp.full((L,), jnp.sum(acc), jnp.float32)
            plsc.store_scatter(outb, [jnp.full((L,), e, jnp.int32)], tot,
                               mask=mask0)
        pltpu.async_copy(outb, out.at[pl.ds(base0 + ci * E, E)], sem_out[b])

    stage_idx(0, 0, sync=True)
    stage_rows(0, 0)
    stage_idx(1, 1)

    def pair_body(p, carry):
        c0 = p * 2
        wait_rows(0)
        stage_idx(0, c0 + 2)
        wait_idx(1, c0 + 1)
        stage_rows(1, c0 + 1)
        compute(0, c0)
        wait_rows(1)
        stage_idx(1, c0 + 3)
        wait_idx(0, c0 + 2)
        stage_rows(0, c0 + 2)
        compute(1, c0 + 1)
        return carry

    lax.fori_loop(0, NCHUNK // 2, pair_body, 0)
    wait_rows(0)
    compute(0, NCHUNK - 1)
    wait_out(0, NCHUNK - 1)
    wait_out(1, NCHUNK - 2)


def kernel(x, edge_index, edge_type, R_diagonal):
    xf = jnp.concatenate([jnp.real(x), jnp.imag(x)], axis=1)
    rf = jnp.concatenate([jnp.real(R_diagonal), jnp.imag(R_diagonal)], axis=1)
    idx_all = jnp.concatenate([edge_index, edge_type[None, :]], axis=0)
    return _sc_score(xf, rf, idx_all)
